# Initial kernel scaffold; baseline (speedup 1.0000x reference)
import jax
import jax.numpy as jnp
from jax.experimental import pallas as pl


def kernel(node_energy, edge_index, node_type, per_type_std, per_type_bias):
    raise NotImplementedError("write your pallas kernel here")


# speedup vs baseline: 224.4118x; 224.4118x over previous
import functools

import jax
import jax.numpy as jnp
from jax import lax
from jax.experimental import pallas as pl
from jax.experimental.pallas import tpu as pltpu
from jax.experimental.pallas import tpu_sc as plsc

L = 16
NC = 2
NS = 16
NW = NC * NS


def _make_mark(n_nodes: int, n_edges: int, chunk: int):
    assert n_edges % NW == 0
    epw = n_edges // NW
    assert epw % chunk == 0 and chunk % L == 0 and chunk % 8 == 0
    assert n_nodes % L == 0
    mesh = plsc.VectorSubcoreMesh(
        core_axis_name="c", subcore_axis_name="s",
        num_cores=NC, num_subcores=NS)

    @functools.partial(
        pl.kernel,
        out_type=jax.ShapeDtypeStruct((NW, n_nodes), jnp.int32),
        mesh=mesh,
        compiler_params=pltpu.CompilerParams(needs_layout_passes=False),
        scratch_types=[
            pltpu.VMEM((n_nodes,), jnp.int32),
            pltpu.VMEM((chunk,), jnp.int32),
        ],
    )
    def mark(edge_src, marker_out, marker_v, idx_v):
        c = lax.axis_index("c")
        s = lax.axis_index("s")
        wid = s * NC + c
        zeros = jnp.zeros((L,), jnp.int32)
        ones = jnp.ones((L,), jnp.int32)

        def zero_body(i, carry):
            marker_v[pl.ds(i * L, L)] = zeros
            return carry
        lax.fori_loop(0, n_nodes // L, zero_body, 0, unroll=8)

        base = wid * epw

        def chunk_body(ci, carry):
            pltpu.sync_copy(
                edge_src.at[pl.ds(base + ci * chunk, chunk)], idx_v)

            def inner(j, icarry):
                idx = idx_v[pl.ds(j * L, L)]
                plsc.store_scatter(marker_v, [idx], ones)
                return icarry
            lax.fori_loop(0, chunk // L, inner, 0, unroll=8)
            return carry
        lax.fori_loop(0, epw // chunk, chunk_body, 0)

        pltpu.sync_copy(marker_v, marker_out.at[wid])

    return mark


def _combine_body(marker_ref, energy_ref, species_ref, std_ref, bias_ref,
                  out_ref):
    num_types = std_ref.shape[0]
    m = jnp.max(marker_ref[...], axis=0) > 0
    species = species_ref[...]
    energy = energy_ref[...]
    scale = jnp.zeros(energy.shape, jnp.float32)
    off = jnp.zeros(energy.shape, jnp.float32)
    for t in range(num_types):
        sel = species == t
        scale = jnp.where(sel, std_ref[t, 0], scale)
        off = jnp.where(sel, bias_ref[t, 0], off)
    out_ref[...] = jnp.where(m, energy * scale + off, energy)


@jax.jit
def kernel(node_energy, edge_index, node_type, per_type_std, per_type_bias):
    n_nodes = node_energy.shape[0]
    n_edges = edge_index.shape[1]

    marker = _make_mark(n_nodes, n_edges, chunk=10000)(edge_index[0])

    out = pl.pallas_call(
        _combine_body,
        out_shape=jax.ShapeDtypeStruct((n_nodes,), jnp.float32),
        in_specs=[
            pl.BlockSpec(memory_space=pltpu.VMEM),
            pl.BlockSpec(memory_space=pltpu.VMEM),
            pl.BlockSpec(memory_space=pltpu.VMEM),
            pl.BlockSpec(memory_space=pltpu.SMEM),
            pl.BlockSpec(memory_space=pltpu.SMEM),
        ],
        out_specs=pl.BlockSpec(memory_space=pltpu.VMEM),
    )(marker, node_energy[:, 0], node_type[:, 0], per_type_std, per_type_bias)
    return out[:, None]


# speedup vs baseline: 463.1737x; 2.0639x over previous
import functools

import jax
import jax.numpy as jnp
from jax import lax
from jax.experimental import pallas as pl
from jax.experimental.pallas import tpu as pltpu
from jax.experimental.pallas import tpu_sc as plsc

L = 16
NC = 2
NS = 16
NW = NC * NS


def _make_mark(n_nodes: int, n_edges: int, chunk: int, nbuf: int):
    assert n_edges % NW == 0
    epw = n_edges // NW
    assert epw % chunk == 0 and chunk % L == 0 and chunk % 8 == 0
    nchunks = epw // chunk
    assert nchunks % nbuf == 0
    n_outer = nchunks // nbuf
    assert n_nodes % L == 0
    mesh = plsc.VectorSubcoreMesh(
        core_axis_name="c", subcore_axis_name="s",
        num_cores=NC, num_subcores=NS)

    @functools.partial(
        pl.kernel,
        out_type=jax.ShapeDtypeStruct((NW, n_nodes), jnp.int32),
        mesh=mesh,
        compiler_params=pltpu.CompilerParams(needs_layout_passes=False),
        scratch_types=[
            pltpu.VMEM((n_nodes,), jnp.int32),
            pltpu.VMEM((nbuf * chunk,), jnp.int32),
        ] + [pltpu.SemaphoreType.DMA] * nbuf,
    )
    def mark(edge_src, marker_out, marker_v, idx_v, *sems):
        c = lax.axis_index("c")
        s = lax.axis_index("s")
        wid = s * NC + c
        zeros = jnp.zeros((L,), jnp.int32)
        ones = jnp.ones((L,), jnp.int32)
        base = wid * epw

        for b in range(nbuf):
            pltpu.async_copy(
                edge_src.at[pl.ds(base + b * chunk, chunk)],
                idx_v.at[pl.ds(b * chunk, chunk)], sems[b])

        @plsc.parallel_loop(0, n_nodes, step=L, unroll=8)
        def _zero(i):
            marker_v[pl.ds(i, L)] = zeros

        def outer(oi, carry):
            for b in range(nbuf):
                pltpu.make_async_copy(
                    edge_src.at[pl.ds(base + (oi * nbuf + b) * chunk, chunk)],
                    idx_v.at[pl.ds(b * chunk, chunk)], sems[b]).wait()

                @plsc.parallel_loop(0, chunk, step=L, unroll=8)
                def _scatter(j):
                    idx = idx_v[pl.ds(b * chunk + j, L)]
                    plsc.store_scatter(marker_v, [idx], ones)

                @pl.when(oi < n_outer - 1)
                def _start_next():
                    pltpu.async_copy(
                        edge_src.at[pl.ds(
                            base + ((oi + 1) * nbuf + b) * chunk, chunk)],
                        idx_v.at[pl.ds(b * chunk, chunk)], sems[b])
            return carry
        lax.fori_loop(0, n_outer, outer, 0)

        pltpu.sync_copy(marker_v, marker_out.at[wid])

    return mark


def _combine_body(marker_ref, energy_ref, species_ref, std_ref, bias_ref,
                  out_ref):
    num_types = std_ref.shape[0]
    m = jnp.max(marker_ref[...], axis=0) > 0
    species = species_ref[...]
    energy = energy_ref[...]
    scale = jnp.zeros(energy.shape, jnp.float32)
    off = jnp.zeros(energy.shape, jnp.float32)
    for t in range(num_types):
        sel = species == t
        scale = jnp.where(sel, std_ref[t, 0], scale)
        off = jnp.where(sel, bias_ref[t, 0], off)
    out_ref[...] = jnp.where(m, energy * scale + off, energy)


@jax.jit
def kernel(node_energy, edge_index, node_type, per_type_std, per_type_bias):
    n_nodes = node_energy.shape[0]
    n_edges = edge_index.shape[1]

    marker = _make_mark(n_nodes, n_edges, chunk=4000, nbuf=5)(edge_index[0])

    out = pl.pallas_call(
        _combine_body,
        out_shape=jax.ShapeDtypeStruct((n_nodes,), jnp.float32),
        in_specs=[
            pl.BlockSpec(memory_space=pltpu.VMEM),
            pl.BlockSpec(memory_space=pltpu.VMEM),
            pl.BlockSpec(memory_space=pltpu.VMEM),
            pl.BlockSpec(memory_space=pltpu.SMEM),
            pl.BlockSpec(memory_space=pltpu.SMEM),
        ],
        out_specs=pl.BlockSpec(memory_space=pltpu.VMEM),
    )(marker, node_energy[:, 0], node_type[:, 0], per_type_std, per_type_bias)
    return out[:, None]


# speedup vs baseline: 490.1528x; 1.0582x over previous
import functools

import jax
import jax.numpy as jnp
from jax import lax
from jax.experimental import pallas as pl
from jax.experimental.pallas import tpu as pltpu
from jax.experimental.pallas import tpu_sc as plsc

L = 16
NC = 2
NS = 16
NW = NC * NS


def _make_mark(n_nodes: int, n_edges: int):
    BLK = 128
    CHUNK = 2560
    NBUF = 3
    assert n_edges % BLK == 0 and CHUNK % BLK == 0 and CHUNK % L == 0
    total_chunks = n_edges // CHUNK
    rounds = total_chunks // NW
    tail = total_chunks - rounds * NW
    assert rounds % NBUF == 0
    n_outer = rounds // NBUF
    assert n_nodes % L == 0
    mesh = plsc.VectorSubcoreMesh(
        core_axis_name="c", subcore_axis_name="s",
        num_cores=NC, num_subcores=NS)

    @functools.partial(
        pl.kernel,
        out_type=jax.ShapeDtypeStruct((NW, n_nodes), jnp.int32),
        mesh=mesh,
        compiler_params=pltpu.CompilerParams(needs_layout_passes=False),
        scratch_types=[
            pltpu.VMEM((n_nodes,), jnp.int32),
        ] + [pltpu.VMEM((2, CHUNK), jnp.int32)] * NBUF
          + [pltpu.SemaphoreType.DMA] * NBUF,
    )
    def mark(edge_index, marker_out, marker_v, *bufs_and_sems):
        idx_v = bufs_and_sems[:NBUF]
        sems = bufs_and_sems[NBUF:]
        c = lax.axis_index("c")
        s = lax.axis_index("s")
        wid = s * NC + c
        zeros = jnp.zeros((L,), jnp.int32)
        ones = jnp.ones((L,), jnp.int32)

        def start(r, b):
            col0 = (r * NW + wid) * CHUNK
            pltpu.async_copy(
                edge_index.at[:, pl.ds(col0, CHUNK)], idx_v[b], sems[b])

        def wait(r, b):
            col0 = (r * NW + wid) * CHUNK
            pltpu.make_async_copy(
                edge_index.at[:, pl.ds(col0, CHUNK)], idx_v[b],
                sems[b]).wait()

        def scatter(b):
            buf = idx_v[b]

            @plsc.parallel_loop(0, CHUNK, step=L, unroll=8)
            def _scatter(j):
                idx = buf[0, pl.ds(j, L)]
                plsc.store_scatter(marker_v, [idx], ones)

        for b in range(NBUF):
            start(b, b)

        @plsc.parallel_loop(0, n_nodes, step=L, unroll=8)
        def _zero(i):
            marker_v[pl.ds(i, L)] = zeros

        def outer(oi, carry):
            for b in range(NBUF):
                r = oi * NBUF + b
                wait(r, b)
                scatter(b)

                @pl.when(oi < n_outer - 1)
                def _start_next():
                    start(r + NBUF, b)
            return carry
        lax.fori_loop(0, n_outer, outer, 0)

        if tail:
            @pl.when(wid < tail)
            def _tail():
                col0 = (rounds * NW + wid) * CHUNK
                pltpu.sync_copy(
                    edge_index.at[:, pl.ds(col0, CHUNK)], idx_v[0])
                scatter(0)

        pltpu.sync_copy(marker_v, marker_out.at[wid])

    return mark


def _combine_body(marker_ref, energy_ref, species_ref, std_ref, bias_ref,
                  out_ref):
    num_types = std_ref.shape[0]
    m = jnp.max(marker_ref[...], axis=0) > 0
    species = species_ref[...]
    energy = energy_ref[...]
    scale = jnp.zeros(energy.shape, jnp.float32)
    off = jnp.zeros(energy.shape, jnp.float32)
    for t in range(num_types):
        sel = species == t
        scale = jnp.where(sel, std_ref[t, 0], scale)
        off = jnp.where(sel, bias_ref[t, 0], off)
    out_ref[...] = jnp.where(m, energy * scale + off, energy)


@jax.jit
def kernel(node_energy, edge_index, node_type, per_type_std, per_type_bias):
    n_nodes = node_energy.shape[0]
    n_edges = edge_index.shape[1]

    marker = _make_mark(n_nodes, n_edges)(edge_index)

    out = pl.pallas_call(
        _combine_body,
        out_shape=jax.ShapeDtypeStruct((n_nodes,), jnp.float32),
        in_specs=[
            pl.BlockSpec(memory_space=pltpu.VMEM),
            pl.BlockSpec(memory_space=pltpu.VMEM),
            pl.BlockSpec(memory_space=pltpu.VMEM),
            pl.BlockSpec(memory_space=pltpu.SMEM),
            pl.BlockSpec(memory_space=pltpu.SMEM),
        ],
        out_specs=pl.BlockSpec(memory_space=pltpu.VMEM),
    )(marker, node_energy[:, 0], node_type[:, 0], per_type_std, per_type_bias)
    return out[:, None]


# speedup vs baseline: 538.5651x; 1.0988x over previous
import functools

import jax
import jax.numpy as jnp
from jax import lax
from jax.experimental import pallas as pl
from jax.experimental.pallas import tpu as pltpu
from jax.experimental.pallas import tpu_sc as plsc

L = 16
NC = 2
NS = 16
NW = NC * NS


def _make_mark(n_nodes: int, n_edges: int):
    BLK = 128
    CHUNK = 2560
    NBUF = 3
    assert n_edges % BLK == 0 and CHUNK % BLK == 0 and CHUNK % L == 0
    total_chunks = n_edges // CHUNK
    rounds = total_chunks // NW
    tail = total_chunks - rounds * NW
    assert rounds % NBUF == 0
    n_outer = rounds // NBUF
    assert n_nodes % L == 0
    mesh = plsc.VectorSubcoreMesh(
        core_axis_name="c", subcore_axis_name="s",
        num_cores=NC, num_subcores=NS)

    @functools.partial(
        pl.kernel,
        out_type=jax.ShapeDtypeStruct((NW, n_nodes), jnp.int32),
        mesh=mesh,
        compiler_params=pltpu.CompilerParams(needs_layout_passes=False),
        scratch_types=[
            pltpu.VMEM((n_nodes,), jnp.int32),
        ] + [pltpu.VMEM((1, CHUNK), jnp.int32)] * NBUF
          + [pltpu.SemaphoreType.DMA] * NBUF,
    )
    def mark(edge_index, marker_out, marker_v, *bufs_and_sems):
        idx_v = bufs_and_sems[:NBUF]
        sems = bufs_and_sems[NBUF:]
        c = lax.axis_index("c")
        s = lax.axis_index("s")
        wid = s * NC + c
        zeros = jnp.zeros((L,), jnp.int32)
        ones = jnp.ones((L,), jnp.int32)

        def start(r, b):
            col0 = (r * NW + wid) * CHUNK
            pltpu.async_copy(
                edge_index.at[pl.ds(0, 1), pl.ds(col0, CHUNK)], idx_v[b],
                sems[b])

        def wait(r, b):
            col0 = (r * NW + wid) * CHUNK
            pltpu.make_async_copy(
                edge_index.at[pl.ds(0, 1), pl.ds(col0, CHUNK)], idx_v[b],
                sems[b]).wait()

        def scatter(b):
            buf = idx_v[b]

            @plsc.parallel_loop(0, CHUNK, step=L, unroll=8)
            def _scatter(j):
                idx = buf[0, pl.ds(j, L)]
                plsc.store_scatter(marker_v, [idx], ones)

        for b in range(NBUF):
            start(b, b)

        @plsc.parallel_loop(0, n_nodes, step=L, unroll=8)
        def _zero(i):
            marker_v[pl.ds(i, L)] = zeros

        def outer(oi, carry):
            for b in range(NBUF):
                r = oi * NBUF + b
                wait(r, b)
                scatter(b)

                @pl.when(oi < n_outer - 1)
                def _start_next():
                    start(r + NBUF, b)
            return carry
        lax.fori_loop(0, n_outer, outer, 0)

        if tail:
            @pl.when(wid < tail)
            def _tail():
                col0 = (rounds * NW + wid) * CHUNK
                pltpu.sync_copy(
                    edge_index.at[pl.ds(0, 1), pl.ds(col0, CHUNK)], idx_v[0])
                scatter(0)

        pltpu.sync_copy(marker_v, marker_out.at[wid])

    return mark


def _combine_body(marker_ref, energy_ref, species_ref, std_ref, bias_ref,
                  out_ref):
    num_types = std_ref.shape[0]
    m = jnp.max(marker_ref[...], axis=0) > 0
    species = species_ref[...]
    energy = energy_ref[...]
    scale = jnp.zeros(energy.shape, jnp.float32)
    off = jnp.zeros(energy.shape, jnp.float32)
    for t in range(num_types):
        sel = species == t
        scale = jnp.where(sel, std_ref[t, 0], scale)
        off = jnp.where(sel, bias_ref[t, 0], off)
    out_ref[...] = jnp.where(m, energy * scale + off, energy)


@jax.jit
def kernel(node_energy, edge_index, node_type, per_type_std, per_type_bias):
    n_nodes = node_energy.shape[0]
    n_edges = edge_index.shape[1]

    marker = _make_mark(n_nodes, n_edges)(edge_index)

    out = pl.pallas_call(
        _combine_body,
        out_shape=jax.ShapeDtypeStruct((n_nodes,), jnp.float32),
        in_specs=[
            pl.BlockSpec(memory_space=pltpu.VMEM),
            pl.BlockSpec(memory_space=pltpu.VMEM),
            pl.BlockSpec(memory_space=pltpu.VMEM),
            pl.BlockSpec(memory_space=pltpu.SMEM),
            pl.BlockSpec(memory_space=pltpu.SMEM),
        ],
        out_specs=pl.BlockSpec(memory_space=pltpu.VMEM),
    )(marker, node_energy[:, 0], node_type[:, 0], per_type_std, per_type_bias)
    return out[:, None]


# speedup vs baseline: 615.5461x; 1.1429x over previous
import functools

import jax
import jax.numpy as jnp
from jax import lax
from jax.experimental import pallas as pl
from jax.experimental.pallas import tpu as pltpu
from jax.experimental.pallas import tpu_sc as plsc

L = 16
NC = 2
NS = 16
NW = NC * NS


def _make_mark(n_nodes: int, n_edges: int):
    BLK = 128
    CHUNK = 5120
    NBUF = 3
    assert n_edges % BLK == 0 and CHUNK % BLK == 0 and CHUNK % L == 0
    total_chunks = n_edges // CHUNK
    rounds = total_chunks // NW
    tail = total_chunks - rounds * NW
    assert rounds % NBUF == 0
    n_outer = rounds // NBUF
    assert n_nodes % L == 0
    mesh = plsc.VectorSubcoreMesh(
        core_axis_name="c", subcore_axis_name="s",
        num_cores=NC, num_subcores=NS)

    @functools.partial(
        pl.kernel,
        out_type=jax.ShapeDtypeStruct((NW, n_nodes), jnp.int32),
        mesh=mesh,
        compiler_params=pltpu.CompilerParams(needs_layout_passes=False),
        scratch_types=[
            pltpu.VMEM((n_nodes,), jnp.int32),
        ] + [pltpu.VMEM((1, CHUNK), jnp.int32)] * NBUF
          + [pltpu.SemaphoreType.DMA] * NBUF,
    )
    def mark(edge_index, marker_out, marker_v, *bufs_and_sems):
        idx_v = bufs_and_sems[:NBUF]
        sems = bufs_and_sems[NBUF:]
        c = lax.axis_index("c")
        s = lax.axis_index("s")
        wid = s * NC + c
        zeros = jnp.zeros((L,), jnp.int32)
        ones = jnp.ones((L,), jnp.int32)

        def start(r, b):
            col0 = (r * NW + wid) * CHUNK
            pltpu.async_copy(
                edge_index.at[pl.ds(0, 1), pl.ds(col0, CHUNK)], idx_v[b],
                sems[b])

        def wait(r, b):
            col0 = (r * NW + wid) * CHUNK
            pltpu.make_async_copy(
                edge_index.at[pl.ds(0, 1), pl.ds(col0, CHUNK)], idx_v[b],
                sems[b]).wait()

        def scatter(b):
            buf = idx_v[b]

            @plsc.parallel_loop(0, CHUNK, step=L, unroll=8)
            def _scatter(j):
                idx = buf[0, pl.ds(j, L)]
                plsc.store_scatter(marker_v, [idx], ones)

        for b in range(NBUF):
            start(b, b)

        @plsc.parallel_loop(0, n_nodes, step=L, unroll=8)
        def _zero(i):
            marker_v[pl.ds(i, L)] = zeros

        def outer(oi, carry):
            for b in range(NBUF):
                r = oi * NBUF + b
                wait(r, b)
                scatter(b)

                @pl.when(oi < n_outer - 1)
                def _start_next():
                    start(r + NBUF, b)
            return carry
        lax.fori_loop(0, n_outer, outer, 0)

        if tail:
            @pl.when(wid < tail)
            def _tail():
                col0 = (rounds * NW + wid) * CHUNK
                pltpu.sync_copy(
                    edge_index.at[pl.ds(0, 1), pl.ds(col0, CHUNK)], idx_v[0])
                scatter(0)

        pltpu.sync_copy(marker_v, marker_out.at[wid])

    return mark


def _combine_body(marker_ref, energy_ref, species_ref, std_ref, bias_ref,
                  out_ref):
    num_types = std_ref.shape[0]
    m = jnp.max(marker_ref[...], axis=0) > 0
    species = species_ref[...]
    energy = energy_ref[...]
    scale = jnp.zeros(energy.shape, jnp.float32)
    off = jnp.zeros(energy.shape, jnp.float32)
    for t in range(num_types):
        sel = species == t
        scale = jnp.where(sel, std_ref[t, 0], scale)
        off = jnp.where(sel, bias_ref[t, 0], off)
    out_ref[...] = jnp.where(m, energy * scale + off, energy)


@jax.jit
def kernel(node_energy, edge_index, node_type, per_type_std, per_type_bias):
    n_nodes = node_energy.shape[0]
    n_edges = edge_index.shape[1]

    marker = _make_mark(n_nodes, n_edges)(edge_index)

    out = pl.pallas_call(
        _combine_body,
        out_shape=jax.ShapeDtypeStruct((n_nodes,), jnp.float32),
        in_specs=[
            pl.BlockSpec(memory_space=pltpu.VMEM),
            pl.BlockSpec(memory_space=pltpu.VMEM),
            pl.BlockSpec(memory_space=pltpu.VMEM),
            pl.BlockSpec(memory_space=pltpu.SMEM),
            pl.BlockSpec(memory_space=pltpu.SMEM),
        ],
        out_specs=pl.BlockSpec(memory_space=pltpu.VMEM),
    )(marker, node_energy[:, 0], node_type[:, 0], per_type_std, per_type_bias)
    return out[:, None]
